# Initial kernel scaffold; baseline (speedup 1.0000x reference)
import jax
import jax.numpy as jnp
from jax.experimental import pallas as pl


def kernel(x, edge_index, avg, mask, W_pre, b_pre, Ws_res, bs_res, W_final, b_final):
    raise NotImplementedError("write your pallas kernel here")


# speedup vs baseline: 2.3063x; 2.3063x over previous
import functools

import jax
import jax.numpy as jnp
from jax.experimental import pallas as pl

N_NODES = 215820
D_IN = 32
DEPTH = 5
OUT_CH = 48


def _mm_body(x_ref, w_ref, o_ref):
    o_ref[...] = jnp.dot(x_ref[...], w_ref[...], preferred_element_type=jnp.float32)


def _mm(a, w):
    n, k = a.shape
    m = w.shape[1]
    bn = 2048
    npad = (n + bn - 1) // bn * bn
    a = jnp.pad(a, ((0, npad - n), (0, 0)))
    out = pl.pallas_call(
        _mm_body,
        grid=(npad // bn,),
        in_specs=[
            pl.BlockSpec((bn, k), lambda i: (i, 0)),
            pl.BlockSpec((k, m), lambda i: (0, 0)),
        ],
        out_specs=pl.BlockSpec((bn, m), lambda i: (i, 0)),
        out_shape=jax.ShapeDtypeStruct((npad, m), jnp.float32),
    )(a, w)
    return out[:n]


def _conv(y, src, dst, dinv, xw, b):
    msg = y[src]
    agg = jnp.zeros_like(y).at[dst].add(msg)
    return dinv[:, None] * agg + dinv[:, None] * y + b


def kernel(x, edge_index, avg, mask, W_pre, b_pre, Ws_res, bs_res, W_final, b_final):
    src = edge_index[0]
    dst = edge_index[1]
    keep = (src != dst).astype(x.dtype)
    ones = jnp.ones_like(keep)
    deg_all = jnp.zeros((N_NODES,), x.dtype).at[dst].add(ones) + 1.0
    deg_keep = jnp.zeros((N_NODES,), x.dtype).at[dst].add(keep) + 1.0
    dinv_a = deg_all ** -0.5
    dinv_k = deg_keep ** -0.5

    xw = _mm(x, W_pre)
    y = dinv_k[:, None] * xw
    msg = y[src] * keep[:, None]
    agg = jnp.zeros_like(y).at[dst].add(msg)
    x1 = dinv_k[:, None] * agg + dinv_k[:, None] * y + b_pre

    h = jnp.where(x != 0, x, x1)
    for i in range(DEPTH):
        xw = _mm(h, Ws_res[i])
        y = dinv_a[:, None] * xw
        h = jax.nn.relu(_conv(y, src, dst, dinv_a, xw, bs_res[i])) + h

    hc = jnp.concatenate([h, x], axis=1)
    xw = _mm(hc, W_final)
    y = dinv_a[:, None] * xw
    out = _conv(y, src, dst, dinv_a, xw, b_final)
    out = jax.nn.sigmoid(out) * 255.0
    return (out * mask[:, None])[None]


# speedup vs baseline: 7.2756x; 3.1546x over previous
import functools

import jax
import jax.numpy as jnp
from jax import lax
from jax.experimental import pallas as pl
from jax.experimental.pallas import tpu as pltpu
from jax.experimental.pallas import tpu_sc as plsc

N_NODES = 215820
D_IN = 32
DEPTH = 5
OUT_CH = 48

NC = 2
NS = 16
NW = NC * NS

E = 1726560
K_EDGE = 1024
CPW = -(-E // (NW * K_EDGE))
E_PAD = NW * K_EDGE * CPW
CPS = E_PAD // (NS * K_EDGE)
ACC_N = 216064
GARB = ACC_N - 1
ZSS = ACC_N // NS
ZB_R = 844
L = 16


def _mesh():
    return plsc.VectorSubcoreMesh(core_axis_name="c", subcore_axis_name="s",
                                  num_cores=NC, num_subcores=NS)


def _deg_body(dst_hbm, dmask_hbm, out_hbm, dst_v, idx2_v, ones_v,
              zero_v, acc_a, acc_k):
    c = lax.axis_index("c")
    s = lax.axis_index("s")
    wid = s * NC + c

    def ofill(i, _):
        ones_v[pl.ds(i * L, L)] = jnp.ones((L,), jnp.float32)
        return 0

    lax.fori_loop(0, K_EDGE // L, ofill, 0)

    def zfill(i, _):
        zero_v[pl.ds(i * L, L)] = jnp.zeros((L,), jnp.float32)
        return 0

    lax.fori_loop(0, ZSS // L, zfill, 0)
    pltpu.sync_copy(zero_v, acc_a.at[pl.ds(s * ZSS, ZSS)])
    pltpu.sync_copy(zero_v, acc_k.at[pl.ds(s * ZSS, ZSS)])
    plsc.subcore_barrier()

    def chunk(j, _):
        base = (wid * CPW + j) * K_EDGE
        pltpu.sync_copy(dst_hbm.at[pl.ds(base, K_EDGE)], dst_v)
        pltpu.sync_copy(dmask_hbm.at[pl.ds(base, K_EDGE)], idx2_v)
        pltpu.sync_copy(ones_v, acc_a.at[dst_v], add=True)
        pltpu.sync_copy(ones_v, acc_k.at[idx2_v], add=True)
        return 0

    lax.fori_loop(0, CPW, chunk, 0)
    plsc.subcore_barrier()
    sl = pl.ds(s * ZSS, ZSS)
    pltpu.sync_copy(acc_a.at[sl], zero_v)
    pltpu.sync_copy(zero_v, out_hbm.at[pl.ds((c * 2) * ACC_N + s * ZSS, ZSS)])
    pltpu.sync_copy(acc_k.at[sl], zero_v)
    pltpu.sync_copy(zero_v, out_hbm.at[pl.ds((c * 2 + 1) * ACC_N + s * ZSS, ZSS)])


def _sc_degrees(dpad, dmask):
    f = pl.kernel(
        _deg_body,
        out_type=jax.ShapeDtypeStruct((NC * 2 * ACC_N,), jnp.float32),
        mesh=_mesh(),
        scratch_types=[
            pltpu.VMEM((K_EDGE,), jnp.int32),
            pltpu.VMEM((K_EDGE,), jnp.int32),
            pltpu.VMEM((K_EDGE,), jnp.float32),
            pltpu.VMEM((ZSS,), jnp.float32),
            pltpu.VMEM_SHARED((ACC_N,), jnp.float32),
            pltpu.VMEM_SHARED((ACC_N,), jnp.float32),
        ],
    )
    return f(dpad, dmask).reshape(NC * 2, ACC_N)


def _conv_body(q, tab_hbm, gidx_hbm, sidx_hbm, zeros_hbm, out_hbm,
               gv, sv, rows_v, bounce_v, sem, acc):
    c = lax.axis_index("c")
    s = lax.axis_index("s")
    rq = q // NC

    def round_body(r, _):
        qi = r * NC + c
        pltpu.sync_copy(zeros_hbm, bounce_v)

        def zcopy(p, _):
            pltpu.sync_copy(bounce_v, acc.at[pl.ds(s * ZSS + p * ZB_R, ZB_R), :])
            return 0

        lax.fori_loop(0, ZSS // ZB_R, zcopy, 0)
        plsc.subcore_barrier()

        def chunk(j, _):
            base = (s * CPS + j) * K_EDGE
            pltpu.sync_copy(gidx_hbm.at[pl.ds(qi * E_PAD + base, K_EDGE)], gv)
            pltpu.sync_copy(sidx_hbm.at[pl.ds(base, K_EDGE)], sv)
            pltpu.async_copy(tab_hbm.at[gv], rows_v, sem).wait()
            pltpu.sync_copy(rows_v, acc.at[sv], add=True)
            return 0

        lax.fori_loop(0, CPS, chunk, 0)
        plsc.subcore_barrier()

        def dump(p, _):
            roff = s * ZSS + p * ZB_R
            pltpu.sync_copy(acc.at[pl.ds(roff, ZB_R), :], bounce_v)
            pltpu.sync_copy(bounce_v,
                            out_hbm.at[pl.ds(qi * ACC_N + roff, ZB_R), :])
            return 0

        lax.fori_loop(0, ZSS // ZB_R, dump, 0)
        plsc.subcore_barrier()
        return 0

    lax.fori_loop(0, rq, round_body, 0)


def _sc_conv(tab, gidx, sidx, zeros_hbm, q):
    f = pl.kernel(
        functools.partial(_conv_body, q),
        out_type=jax.ShapeDtypeStruct((q * ACC_N, 8), jnp.float32),
        mesh=_mesh(),
        scratch_types=[
            pltpu.VMEM((K_EDGE,), jnp.int32),
            pltpu.VMEM((K_EDGE,), jnp.int32),
            pltpu.VMEM((K_EDGE, 8), jnp.float32),
            pltpu.VMEM((ZB_R, 8), jnp.float32),
            pltpu.SemaphoreType.DMA,
            pltpu.VMEM_SHARED((ACC_N, 8), jnp.float32),
        ],
        compiler_params=pltpu.CompilerParams(use_tc_tiling_on_sc=False),
    )
    return f(tab, gidx, sidx, zeros_hbm)


def _mm_body(x_ref, w_ref, o_ref):
    o_ref[...] = jnp.dot(x_ref[...], w_ref[...], preferred_element_type=jnp.float32)


def _mm(a, w):
    n, k = a.shape
    m = w.shape[1]
    bn = 2048
    npad = (n + bn - 1) // bn * bn
    a = jnp.pad(a, ((0, npad - n), (0, 0)))
    out = pl.pallas_call(
        _mm_body,
        grid=(npad // bn,),
        in_specs=[
            pl.BlockSpec((bn, k), lambda i: (i, 0)),
            pl.BlockSpec((k, m), lambda i: (0, 0)),
        ],
        out_specs=pl.BlockSpec((bn, m), lambda i: (i, 0)),
        out_shape=jax.ShapeDtypeStruct((npad, m), jnp.float32),
    )(a, w)
    return out[:n]


def _gcn_agg(y, gidx, sidx, zeros_hbm, q):
    tab = y.reshape(N_NODES, q, 8).transpose(1, 0, 2).reshape(q * N_NODES, 8)
    outf = _sc_conv(tab, gidx, sidx, zeros_hbm, q)
    return (outf.reshape(q, ACC_N, 8)[:, :N_NODES]
            .transpose(1, 0, 2).reshape(N_NODES, q * 8))


def kernel(x, edge_index, avg, mask, W_pre, b_pre, Ws_res, bs_res, W_final, b_final):
    src = edge_index[0]
    dst = edge_index[1]
    spad = jnp.pad(src, (0, E_PAD - E))
    dpad = jnp.pad(dst, (0, E_PAD - E), constant_values=GARB)
    dmask = jnp.where(spad == dpad, GARB, dpad)
    gidx = (spad[None, :] + (jnp.arange(6, dtype=jnp.int32) * N_NODES)[:, None]
            ).reshape(-1)
    zeros_hbm = jnp.zeros((ZB_R, 8), jnp.float32)

    parts = _sc_degrees(dpad, dmask)
    deg_all = parts[0, :N_NODES] + parts[2, :N_NODES] + 1.0
    deg_keep = parts[1, :N_NODES] + parts[3, :N_NODES] + 1.0
    dinv_a = (deg_all ** -0.5)[:, None]
    dinv_k = (deg_keep ** -0.5)[:, None]

    xw = _mm(x, W_pre)
    y = dinv_k * xw
    agg = _gcn_agg(y, gidx, dmask, zeros_hbm, 4)
    x1 = dinv_k * agg + dinv_k * y + b_pre

    h = jnp.where(x != 0, x, x1)
    for i in range(DEPTH):
        xw = _mm(h, Ws_res[i])
        y = dinv_a * xw
        agg = _gcn_agg(y, gidx, dpad, zeros_hbm, 4)
        h = jax.nn.relu(dinv_a * agg + dinv_a * y + bs_res[i]) + h

    hc = jnp.concatenate([h, x], axis=1)
    xw = _mm(hc, W_final)
    y = dinv_a * xw
    agg = _gcn_agg(y, gidx, dpad, zeros_hbm, 6)
    out = dinv_a * agg + dinv_a * y + b_final
    out = jax.nn.sigmoid(out) * 255.0
    return (out * mask[:, None])[None]


# speedup vs baseline: 8.0910x; 1.1121x over previous
import functools

import jax
import jax.numpy as jnp
from jax import lax
from jax.experimental import pallas as pl
from jax.experimental.pallas import tpu as pltpu
from jax.experimental.pallas import tpu_sc as plsc

N_NODES = 215820
D_IN = 32
DEPTH = 5
OUT_CH = 48

NC = 2
NS = 16
NW = NC * NS

E = 1726560
K_EDGE = 1024
CPW = -(-E // (NW * K_EDGE))
E_PAD = NW * K_EDGE * CPW
CPS = E_PAD // (NS * K_EDGE)
ACC_N = 216064
GARB = ACC_N - 1
ZSS = ACC_N // NS
ZB_R = 211
L = 16


def _mesh():
    return plsc.VectorSubcoreMesh(core_axis_name="c", subcore_axis_name="s",
                                  num_cores=NC, num_subcores=NS)


def _deg_body(dst_hbm, dmask_hbm, out_hbm, dst_v, idx2_v, ones_v,
              zero_v, acc_a, acc_k):
    c = lax.axis_index("c")
    s = lax.axis_index("s")
    wid = s * NC + c

    def ofill(i, _):
        ones_v[pl.ds(i * L, L)] = jnp.ones((L,), jnp.float32)
        return 0

    lax.fori_loop(0, K_EDGE // L, ofill, 0)

    def zfill(i, _):
        zero_v[pl.ds(i * L, L)] = jnp.zeros((L,), jnp.float32)
        return 0

    lax.fori_loop(0, ZSS // L, zfill, 0)
    pltpu.sync_copy(zero_v, acc_a.at[pl.ds(s * ZSS, ZSS)])
    pltpu.sync_copy(zero_v, acc_k.at[pl.ds(s * ZSS, ZSS)])
    plsc.subcore_barrier()

    def chunk(j, _):
        base = (wid * CPW + j) * K_EDGE
        pltpu.sync_copy(dst_hbm.at[pl.ds(base, K_EDGE)], dst_v)
        pltpu.sync_copy(dmask_hbm.at[pl.ds(base, K_EDGE)], idx2_v)
        pltpu.sync_copy(ones_v, acc_a.at[dst_v], add=True)
        pltpu.sync_copy(ones_v, acc_k.at[idx2_v], add=True)
        return 0

    lax.fori_loop(0, CPW, chunk, 0)
    plsc.subcore_barrier()
    sl = pl.ds(s * ZSS, ZSS)
    pltpu.sync_copy(acc_a.at[sl], zero_v)
    pltpu.sync_copy(zero_v, out_hbm.at[pl.ds((c * 2) * ACC_N + s * ZSS, ZSS)])
    pltpu.sync_copy(acc_k.at[sl], zero_v)
    pltpu.sync_copy(zero_v, out_hbm.at[pl.ds((c * 2 + 1) * ACC_N + s * ZSS, ZSS)])


def _sc_degrees(dpad, dmask):
    f = pl.kernel(
        _deg_body,
        out_type=jax.ShapeDtypeStruct((NC * 2 * ACC_N,), jnp.float32),
        mesh=_mesh(),
        scratch_types=[
            pltpu.VMEM((K_EDGE,), jnp.int32),
            pltpu.VMEM((K_EDGE,), jnp.int32),
            pltpu.VMEM((K_EDGE,), jnp.float32),
            pltpu.VMEM((ZSS,), jnp.float32),
            pltpu.VMEM_SHARED((ACC_N,), jnp.float32),
            pltpu.VMEM_SHARED((ACC_N,), jnp.float32),
        ],
    )
    return f(dpad, dmask).reshape(NC * 2, ACC_N)


def _conv_body(q, tab_hbm, gidx_hbm, sidx_hbm, zeros_hbm, out_hbm,
               gv0, gv1, sv0, sv1, rows0, rows1, bounce_v,
               isem0, isem1, gsem0, gsem1, acc):
    c = lax.axis_index("c")
    s = lax.axis_index("s")
    rq = q // NC
    gv = (gv0, gv1)
    sv = (sv0, sv1)
    rows = (rows0, rows1)
    isem = (isem0, isem1)
    gsem = (gsem0, gsem1)

    def round_body(r, _):
        qi = r * NC + c
        pltpu.sync_copy(zeros_hbm, bounce_v)

        def zcopy(p, _):
            pltpu.sync_copy(bounce_v, acc.at[pl.ds(s * ZSS + p * ZB_R, ZB_R), :])
            return 0

        lax.fori_loop(0, ZSS // ZB_R, zcopy, 0)
        plsc.subcore_barrier()

        def idx_descs(j, b):
            base = (s * CPS + j) * K_EDGE
            return (
                pltpu.make_async_copy(
                    gidx_hbm.at[pl.ds(qi * E_PAD + base, K_EDGE)], gv[b], isem[b]),
                pltpu.make_async_copy(
                    sidx_hbm.at[pl.ds(base, K_EDGE)], sv[b], isem[b]),
            )

        def start_idx(j, b):
            for d in idx_descs(j, b):
                d.start()

        def wait_idx(j, b):
            for d in idx_descs(j, b):
                d.wait()

        def gather_desc(b):
            return pltpu.make_async_copy(tab_hbm.at[gv[b]], rows[b], gsem[b])

        def scatter(b):
            pltpu.sync_copy(rows[b], acc.at[sv[b]], add=True)

        start_idx(0, 0)
        start_idx(1, 1)
        wait_idx(0, 0)
        gather_desc(0).start()

        def pair(g, _):
            for b, joff in ((1, 1), (0, 2)):
                j = 2 * g + joff
                wait_idx(j, b)
                gather_desc(b).start()
                gather_desc(1 - b).wait()
                scatter(1 - b)
                start_idx(j + 1, 1 - b)

            return 0

        lax.fori_loop(0, (CPS - 2) // 2, pair, 0)

        wait_idx(CPS - 1, 1)
        gather_desc(1).start()
        gather_desc(0).wait()
        scatter(0)
        gather_desc(1).wait()
        scatter(1)
        plsc.subcore_barrier()

        def dump(p, _):
            roff = s * ZSS + p * ZB_R
            pltpu.sync_copy(acc.at[pl.ds(roff, ZB_R), :], bounce_v)
            pltpu.sync_copy(bounce_v,
                            out_hbm.at[pl.ds(qi * ACC_N + roff, ZB_R), :])
            return 0

        lax.fori_loop(0, ZSS // ZB_R, dump, 0)
        plsc.subcore_barrier()
        return 0

    lax.fori_loop(0, rq, round_body, 0)


def _sc_conv(tab, gidx, sidx, zeros_hbm, q):
    f = pl.kernel(
        functools.partial(_conv_body, q),
        out_type=jax.ShapeDtypeStruct((q * ACC_N, 8), jnp.float32),
        mesh=_mesh(),
        scratch_types=[
            pltpu.VMEM((K_EDGE,), jnp.int32),
            pltpu.VMEM((K_EDGE,), jnp.int32),
            pltpu.VMEM((K_EDGE,), jnp.int32),
            pltpu.VMEM((K_EDGE,), jnp.int32),
            pltpu.VMEM((K_EDGE, 8), jnp.float32),
            pltpu.VMEM((K_EDGE, 8), jnp.float32),
            pltpu.VMEM((ZB_R, 8), jnp.float32),
            pltpu.SemaphoreType.DMA,
            pltpu.SemaphoreType.DMA,
            pltpu.SemaphoreType.DMA,
            pltpu.SemaphoreType.DMA,
            pltpu.VMEM_SHARED((ACC_N, 8), jnp.float32),
        ],
        compiler_params=pltpu.CompilerParams(use_tc_tiling_on_sc=False),
    )
    return f(tab, gidx, sidx, zeros_hbm)


def _mm_body(x_ref, w_ref, o_ref):
    o_ref[...] = jnp.dot(x_ref[...], w_ref[...], preferred_element_type=jnp.float32)


def _mm(a, w):
    n, k = a.shape
    m = w.shape[1]
    bn = 2048
    npad = (n + bn - 1) // bn * bn
    a = jnp.pad(a, ((0, npad - n), (0, 0)))
    out = pl.pallas_call(
        _mm_body,
        grid=(npad // bn,),
        in_specs=[
            pl.BlockSpec((bn, k), lambda i: (i, 0)),
            pl.BlockSpec((k, m), lambda i: (0, 0)),
        ],
        out_specs=pl.BlockSpec((bn, m), lambda i: (i, 0)),
        out_shape=jax.ShapeDtypeStruct((npad, m), jnp.float32),
    )(a, w)
    return out[:n]


def _gcn_agg(y, gidx, sidx, zeros_hbm, q):
    tab = y.reshape(N_NODES, q, 8).transpose(1, 0, 2).reshape(q * N_NODES, 8)
    outf = _sc_conv(tab, gidx, sidx, zeros_hbm, q)
    return (outf.reshape(q, ACC_N, 8)[:, :N_NODES]
            .transpose(1, 0, 2).reshape(N_NODES, q * 8))


def kernel(x, edge_index, avg, mask, W_pre, b_pre, Ws_res, bs_res, W_final, b_final):
    src = edge_index[0]
    dst = edge_index[1]
    spad = jnp.pad(src, (0, E_PAD - E))
    dpad = jnp.pad(dst, (0, E_PAD - E), constant_values=GARB)
    dmask = jnp.where(spad == dpad, GARB, dpad)
    gidx = (spad[None, :] + (jnp.arange(6, dtype=jnp.int32) * N_NODES)[:, None]
            ).reshape(-1)
    zeros_hbm = jnp.zeros((ZB_R, 8), jnp.float32)

    parts = _sc_degrees(dpad, dmask)
    deg_all = parts[0, :N_NODES] + parts[2, :N_NODES] + 1.0
    deg_keep = parts[1, :N_NODES] + parts[3, :N_NODES] + 1.0
    dinv_a = (deg_all ** -0.5)[:, None]
    dinv_k = (deg_keep ** -0.5)[:, None]

    xw = _mm(x, W_pre)
    y = dinv_k * xw
    agg = _gcn_agg(y, gidx, dmask, zeros_hbm, 4)
    x1 = dinv_k * agg + dinv_k * y + b_pre

    h = jnp.where(x != 0, x, x1)
    for i in range(DEPTH):
        xw = _mm(h, Ws_res[i])
        y = dinv_a * xw
        agg = _gcn_agg(y, gidx, dpad, zeros_hbm, 4)
        h = jax.nn.relu(dinv_a * agg + dinv_a * y + bs_res[i]) + h

    hc = jnp.concatenate([h, x], axis=1)
    xw = _mm(hc, W_final)
    y = dinv_a * xw
    agg = _gcn_agg(y, gidx, dpad, zeros_hbm, 6)
    out = dinv_a * agg + dinv_a * y + b_final
    out = jax.nn.sigmoid(out) * 255.0
    return (out * mask[:, None])[None]


# speedup vs baseline: 13.0109x; 1.6081x over previous
import functools

import jax
import jax.numpy as jnp
from jax import lax
from jax.experimental import pallas as pl
from jax.experimental.pallas import tpu as pltpu
from jax.experimental.pallas import tpu_sc as plsc

N_NODES = 215820
D_IN = 32
DEPTH = 5
OUT_CH = 48

NC = 2
NS = 16
NW = NC * NS
L = 16

E = 1726560
K_EDGE = 848
K_DEG = 4 * K_EDGE
CPW = 16
E_PAD = NW * K_DEG * CPW
CPS = E_PAD // (NS * K_EDGE)
ACC_N = 217088
GARB = ACC_N - 1
ZSS = ACC_N // NS
DMP = ZSS // K_EDGE


def _mesh():
    return plsc.VectorSubcoreMesh(core_axis_name="c", subcore_axis_name="s",
                                  num_cores=NC, num_subcores=NS)


def _deg_body(dst_hbm, dmask_hbm, out_hbm, d0, d1, m0, m1, ones_v, zero_v,
              isem0, isem1, acc_a, acc_k):
    c = lax.axis_index("c")
    s = lax.axis_index("s")
    wid = s * NC + c
    dbuf = (d0, d1)
    mbuf = (m0, m1)
    isem = (isem0, isem1)

    def ofill(i, _):
        ones_v[pl.ds(i * L, L)] = jnp.ones((L,), jnp.float32)
        return 0

    lax.fori_loop(0, K_DEG // L, ofill, 0)

    def zfill(i, _):
        zero_v[pl.ds(i * L, L)] = jnp.zeros((L,), jnp.float32)
        return 0

    lax.fori_loop(0, ZSS // L, zfill, 0)
    pltpu.sync_copy(zero_v, acc_a.at[pl.ds(s * ZSS, ZSS)])
    pltpu.sync_copy(zero_v, acc_k.at[pl.ds(s * ZSS, ZSS)])
    plsc.subcore_barrier()

    def idx_descs(j, b):
        base = (wid * CPW + j) * K_DEG
        return (
            pltpu.make_async_copy(dst_hbm.at[pl.ds(base, K_DEG)], dbuf[b], isem[b]),
            pltpu.make_async_copy(dmask_hbm.at[pl.ds(base, K_DEG)], mbuf[b], isem[b]),
        )

    def start_idx(j, b):
        for d in idx_descs(j, b):
            d.start()

    def wait_idx(j, b):
        for d in idx_descs(j, b):
            d.wait()

    def scatter(b):
        pltpu.sync_copy(ones_v, acc_a.at[dbuf[b]], add=True)
        pltpu.sync_copy(ones_v, acc_k.at[mbuf[b]], add=True)

    start_idx(0, 0)

    def pair(g, _):
        for b, joff in ((0, 0), (1, 1)):
            j = 2 * g + joff
            wait_idx(j, b)
            start_idx(j + 1, 1 - b)
            scatter(b)
        return 0

    lax.fori_loop(0, CPW // 2 - 1, pair, 0)
    wait_idx(CPW - 2, 0)
    start_idx(CPW - 1, 1)
    scatter(0)
    wait_idx(CPW - 1, 1)
    scatter(1)

    plsc.subcore_barrier()
    sl = pl.ds(s * ZSS, ZSS)
    pltpu.sync_copy(acc_a.at[sl], zero_v)
    pltpu.sync_copy(zero_v, out_hbm.at[pl.ds((c * 2) * ACC_N + s * ZSS, ZSS)])
    pltpu.sync_copy(acc_k.at[sl], zero_v)
    pltpu.sync_copy(zero_v, out_hbm.at[pl.ds((c * 2 + 1) * ACC_N + s * ZSS, ZSS)])


def _sc_degrees(dpad, dmask):
    f = pl.kernel(
        _deg_body,
        out_type=jax.ShapeDtypeStruct((NC * 2 * ACC_N,), jnp.float32),
        mesh=_mesh(),
        scratch_types=[
            pltpu.VMEM((K_DEG,), jnp.int32),
            pltpu.VMEM((K_DEG,), jnp.int32),
            pltpu.VMEM((K_DEG,), jnp.int32),
            pltpu.VMEM((K_DEG,), jnp.int32),
            pltpu.VMEM((K_DEG,), jnp.float32),
            pltpu.VMEM((ZSS,), jnp.float32),
            pltpu.SemaphoreType.DMA,
            pltpu.SemaphoreType.DMA,
            pltpu.VMEM_SHARED((ACC_N,), jnp.float32),
            pltpu.VMEM_SHARED((ACC_N,), jnp.float32),
        ],
        compiler_params=pltpu.CompilerParams(use_tc_tiling_on_sc=False),
    )
    return f(dpad, dmask).reshape(NC * 2, ACC_N)


def _conv_body(q, tab_hbm, gidx_hbm, sidx_hbm, oidx_hbm, zeros_hbm, out_hbm,
               gv0, gv1, sv0, sv1, rows0, rows1,
               isem0, isem1, gsem0, gsem1, acc):
    c = lax.axis_index("c")
    s = lax.axis_index("s")
    rq = q // NC
    gv = (gv0, gv1)
    sv = (sv0, sv1)
    rows = (rows0, rows1)
    isem = (isem0, isem1)
    gsem = (gsem0, gsem1)

    def round_body(r, _):
        qi = r * NC + c
        pltpu.sync_copy(zeros_hbm, rows0)

        def zcopy(p, _):
            pltpu.sync_copy(rows0, acc.at[pl.ds(s * ZSS + p * K_EDGE, K_EDGE), :])
            return 0

        lax.fori_loop(0, DMP, zcopy, 0)
        plsc.subcore_barrier()

        def idx_descs(j, b):
            base = (s * CPS + j) * K_EDGE
            return (
                pltpu.make_async_copy(
                    gidx_hbm.at[pl.ds(qi * E_PAD + base, K_EDGE)], gv[b], isem[b]),
                pltpu.make_async_copy(
                    sidx_hbm.at[pl.ds(base, K_EDGE)], sv[b], isem[b]),
            )

        def start_idx(j, b):
            for d in idx_descs(j, b):
                d.start()

        def wait_idx(j, b):
            for d in idx_descs(j, b):
                d.wait()

        def gather_desc(b):
            return pltpu.make_async_copy(tab_hbm.at[gv[b]], rows[b], gsem[b])

        def scatter(b):
            pltpu.sync_copy(rows[b], acc.at[sv[b]], add=True)

        start_idx(0, 0)
        start_idx(1, 1)
        wait_idx(0, 0)
        gather_desc(0).start()

        def pair(g, _):
            for b, joff in ((1, 1), (0, 2)):
                j = 2 * g + joff
                wait_idx(j, b)
                gather_desc(b).start()
                gather_desc(1 - b).wait()
                scatter(1 - b)
                start_idx(j + 1, 1 - b)

            return 0

        lax.fori_loop(0, (CPS - 2) // 2, pair, 0)

        wait_idx(CPS - 1, 1)
        gather_desc(1).start()
        gather_desc(0).wait()
        scatter(0)
        gather_desc(1).wait()
        scatter(1)
        plsc.subcore_barrier()

        def dump(p, _):
            roff = s * ZSS + p * K_EDGE
            pltpu.sync_copy(oidx_hbm.at[pl.ds(qi * ACC_N + roff, K_EDGE)], sv0)
            pltpu.sync_copy(acc.at[pl.ds(roff, K_EDGE), :], rows0)
            pltpu.sync_copy(rows0, out_hbm.at[sv0])
            return 0

        lax.fori_loop(0, DMP, dump, 0)
        plsc.subcore_barrier()
        return 0

    lax.fori_loop(0, rq, round_body, 0)


def _sc_conv(tab, gidx, sidx, oidx, zeros_hbm, q):
    f = pl.kernel(
        functools.partial(_conv_body, q),
        out_type=jax.ShapeDtypeStruct((q * ACC_N, 8), jnp.float32),
        mesh=_mesh(),
        scratch_types=[
            pltpu.VMEM((K_EDGE,), jnp.int32),
            pltpu.VMEM((K_EDGE,), jnp.int32),
            pltpu.VMEM((K_EDGE,), jnp.int32),
            pltpu.VMEM((K_EDGE,), jnp.int32),
            pltpu.VMEM((K_EDGE, 8), jnp.float32),
            pltpu.VMEM((K_EDGE, 8), jnp.float32),
            pltpu.SemaphoreType.DMA,
            pltpu.SemaphoreType.DMA,
            pltpu.SemaphoreType.DMA,
            pltpu.SemaphoreType.DMA,
            pltpu.VMEM_SHARED((ACC_N, 8), jnp.float32),
        ],
        compiler_params=pltpu.CompilerParams(use_tc_tiling_on_sc=False),
    )
    return f(tab, gidx, sidx, oidx, zeros_hbm)


def _mm_body(x_ref, w_ref, o_ref):
    o_ref[...] = jnp.dot(x_ref[...], w_ref[...], preferred_element_type=jnp.float32)


def _mm(a, w):
    n, k = a.shape
    m = w.shape[1]
    bn = 2048
    npad = (n + bn - 1) // bn * bn
    a = jnp.pad(a, ((0, npad - n), (0, 0)))
    out = pl.pallas_call(
        _mm_body,
        grid=(npad // bn,),
        in_specs=[
            pl.BlockSpec((bn, k), lambda i: (i, 0)),
            pl.BlockSpec((k, m), lambda i: (0, 0)),
        ],
        out_specs=pl.BlockSpec((bn, m), lambda i: (i, 0)),
        out_shape=jax.ShapeDtypeStruct((npad, m), jnp.float32),
    )(a, w)
    return out[:n]


def _gcn_agg(y, gidx, sidx, oidx, zeros_hbm, q):
    tab = y.reshape(q * N_NODES, 8)
    outf = _sc_conv(tab, gidx, sidx, oidx, zeros_hbm, q)
    return outf.reshape(q * ACC_N * 8)[:N_NODES * q * 8].reshape(N_NODES, q * 8)


def kernel(x, edge_index, avg, mask, W_pre, b_pre, Ws_res, bs_res, W_final, b_final):
    src = edge_index[0]
    dst = edge_index[1]
    spad = jnp.pad(src, (0, E_PAD - E))
    dpad = jnp.pad(dst, (0, E_PAD - E), constant_values=GARB)
    dmask = jnp.where(spad == dpad, GARB, dpad)
    qs4 = jnp.arange(4, dtype=jnp.int32)
    qs6 = jnp.arange(6, dtype=jnp.int32)
    gidx4 = ((spad * 4)[None, :] + qs4[:, None]).reshape(-1)
    gidx6 = ((spad * 6)[None, :] + qs6[:, None]).reshape(-1)
    ar = jnp.arange(ACC_N, dtype=jnp.int32)
    oidx4 = ((ar * 4)[None, :] + qs4[:, None]).reshape(-1)
    oidx6 = ((ar * 6)[None, :] + qs6[:, None]).reshape(-1)
    zeros_hbm = jnp.zeros((K_EDGE, 8), jnp.float32)

    parts = _sc_degrees(dpad, dmask)
    deg_all = parts[0, :N_NODES] + parts[2, :N_NODES] + 1.0
    deg_keep = parts[1, :N_NODES] + parts[3, :N_NODES] + 1.0
    dinv_a = (deg_all ** -0.5)[:, None]
    dinv_k = (deg_keep ** -0.5)[:, None]

    xw = _mm(x, W_pre)
    y = dinv_k * xw
    agg = _gcn_agg(y, gidx4, dmask, oidx4, zeros_hbm, 4)
    x1 = dinv_k * agg + dinv_k * y + b_pre

    h = jnp.where(x != 0, x, x1)
    for i in range(DEPTH):
        xw = _mm(h, Ws_res[i])
        y = dinv_a * xw
        agg = _gcn_agg(y, gidx4, dpad, oidx4, zeros_hbm, 4)
        h = jax.nn.relu(dinv_a * agg + dinv_a * y + bs_res[i]) + h

    hc = jnp.concatenate([h, x], axis=1)
    xw = _mm(hc, W_final)
    y = dinv_a * xw
    agg = _gcn_agg(y, gidx6, dpad, oidx6, zeros_hbm, 6)
    out = dinv_a * agg + dinv_a * y + b_final
    out = jax.nn.sigmoid(out) * 255.0
    return (out * mask[:, None])[None]


# speedup vs baseline: 13.4475x; 1.0336x over previous
import functools

import jax
import jax.numpy as jnp
from jax import lax
from jax.experimental import pallas as pl
from jax.experimental.pallas import tpu as pltpu
from jax.experimental.pallas import tpu_sc as plsc

N_NODES = 215820
D_IN = 32
DEPTH = 5
OUT_CH = 48

NC = 2
NS = 16
NW = NC * NS
L = 16

E = 1726560
K_EDGE = 848
K_DEG = 4 * K_EDGE
CPW = 16
E_PAD = NW * K_DEG * CPW
CPS = E_PAD // (NS * K_EDGE)
ACC_N = 217088
GARB = ACC_N - 1
ZSS = ACC_N // NS
DS = K_EDGE
DMP = ZSS // DS


def _mesh():
    return plsc.VectorSubcoreMesh(core_axis_name="c", subcore_axis_name="s",
                                  num_cores=NC, num_subcores=NS)


def _deg_body(dst_hbm, dmask_hbm, out_hbm, d0, d1, m0, m1, ones_v, zero_v,
              isem0, isem1, acc_a, acc_k):
    c = lax.axis_index("c")
    s = lax.axis_index("s")
    wid = s * NC + c
    dbuf = (d0, d1)
    mbuf = (m0, m1)
    isem = (isem0, isem1)

    def ofill(i, _):
        ones_v[pl.ds(i * L, L)] = jnp.ones((L,), jnp.float32)
        return 0

    lax.fori_loop(0, K_DEG // L, ofill, 0)

    def zfill(i, _):
        zero_v[pl.ds(i * L, L)] = jnp.zeros((L,), jnp.float32)
        return 0

    lax.fori_loop(0, ZSS // L, zfill, 0)
    pltpu.sync_copy(zero_v, acc_a.at[pl.ds(s * ZSS, ZSS)])
    pltpu.sync_copy(zero_v, acc_k.at[pl.ds(s * ZSS, ZSS)])
    plsc.subcore_barrier()

    def idx_descs(j, b):
        base = (wid * CPW + j) * K_DEG
        return (
            pltpu.make_async_copy(dst_hbm.at[pl.ds(base, K_DEG)], dbuf[b], isem[b]),
            pltpu.make_async_copy(dmask_hbm.at[pl.ds(base, K_DEG)], mbuf[b], isem[b]),
        )

    def start_idx(j, b):
        for d in idx_descs(j, b):
            d.start()

    def wait_idx(j, b):
        for d in idx_descs(j, b):
            d.wait()

    def scatter(b):
        pltpu.sync_copy(ones_v, acc_a.at[dbuf[b]], add=True)
        pltpu.sync_copy(ones_v, acc_k.at[mbuf[b]], add=True)

    start_idx(0, 0)

    def pair(g, _):
        for b, joff in ((0, 0), (1, 1)):
            j = 2 * g + joff
            wait_idx(j, b)
            start_idx(j + 1, 1 - b)
            scatter(b)
        return 0

    lax.fori_loop(0, CPW // 2 - 1, pair, 0)
    wait_idx(CPW - 2, 0)
    start_idx(CPW - 1, 1)
    scatter(0)
    wait_idx(CPW - 1, 1)
    scatter(1)

    plsc.subcore_barrier()
    sl = pl.ds(s * ZSS, ZSS)
    pltpu.sync_copy(acc_a.at[sl], zero_v)
    pltpu.sync_copy(zero_v, out_hbm.at[pl.ds((c * 2) * ACC_N + s * ZSS, ZSS)])
    pltpu.sync_copy(acc_k.at[sl], zero_v)
    pltpu.sync_copy(zero_v, out_hbm.at[pl.ds((c * 2 + 1) * ACC_N + s * ZSS, ZSS)])


def _sc_degrees(dpad, dmask):
    f = pl.kernel(
        _deg_body,
        out_type=jax.ShapeDtypeStruct((NC * 2 * ACC_N,), jnp.float32),
        mesh=_mesh(),
        scratch_types=[
            pltpu.VMEM((K_DEG,), jnp.int32),
            pltpu.VMEM((K_DEG,), jnp.int32),
            pltpu.VMEM((K_DEG,), jnp.int32),
            pltpu.VMEM((K_DEG,), jnp.int32),
            pltpu.VMEM((K_DEG,), jnp.float32),
            pltpu.VMEM((ZSS,), jnp.float32),
            pltpu.SemaphoreType.DMA,
            pltpu.SemaphoreType.DMA,
            pltpu.VMEM_SHARED((ACC_N,), jnp.float32),
            pltpu.VMEM_SHARED((ACC_N,), jnp.float32),
        ],
        compiler_params=pltpu.CompilerParams(use_tc_tiling_on_sc=False),
    )
    return f(dpad, dmask).reshape(NC * 2, ACC_N)


def _conv_body(q, tab_hbm, gidx_hbm, sidx_hbm, oidx_hbm, zeros_hbm, out_hbm,
               gv0, gv1, sv0, sv1, rows0, rows1, dv0, dv1,
               isem0, isem1, gsem0, gsem1, acc):
    c = lax.axis_index("c")
    s = lax.axis_index("s")
    rq = q // NC
    gv = (gv0, gv1)
    sv = (sv0, sv1)
    rows = (rows0, rows1)
    isem = (isem0, isem1)
    gsem = (gsem0, gsem1)

    def round_body(r, _):
        qi = r * NC + c
        pltpu.sync_copy(zeros_hbm, rows0)

        def zcopy(p, _):
            pltpu.sync_copy(rows0, acc.at[pl.ds(s * ZSS + p * DS, DS), :])
            return 0

        lax.fori_loop(0, DMP, zcopy, 0)
        plsc.subcore_barrier()

        def idx_descs(j, b):
            base = (s * CPS + j) * K_EDGE
            return (
                pltpu.make_async_copy(
                    gidx_hbm.at[pl.ds(qi * E_PAD + base, K_EDGE)], gv[b], isem[b]),
                pltpu.make_async_copy(
                    sidx_hbm.at[pl.ds(base, K_EDGE)], sv[b], isem[b]),
            )

        def start_idx(j, b):
            for d in idx_descs(j, b):
                d.start()

        def wait_idx(j, b):
            for d in idx_descs(j, b):
                d.wait()

        def gather_desc(b):
            return pltpu.make_async_copy(tab_hbm.at[gv[b]], rows[b], gsem[b])

        def scatter(b):
            pltpu.sync_copy(rows[b], acc.at[sv[b]], add=True)

        start_idx(0, 0)
        start_idx(1, 1)
        wait_idx(0, 0)
        gather_desc(0).start()

        def pair(g, _):
            for b, joff in ((1, 1), (0, 2)):
                j = 2 * g + joff
                wait_idx(j, b)
                gather_desc(b).start()
                gather_desc(1 - b).wait()
                scatter(1 - b)
                start_idx(j + 1, 1 - b)

            return 0

        lax.fori_loop(0, (CPS - 2) // 2, pair, 0)

        wait_idx(CPS - 1, 1)
        gather_desc(1).start()
        gather_desc(0).wait()
        scatter(0)
        gather_desc(1).wait()
        scatter(1)
        plsc.subcore_barrier()

        dv = (dv0, dv1)

        def dump_desc(p, b):
            roff = s * ZSS + p * DS
            return pltpu.make_async_copy(
                oidx_hbm.at[pl.ds(qi * ACC_N + roff, DS)], dv[b], isem[b])

        def dump_start(p, b):
            dump_desc(p, b).start()

        def dump_wait(p, b):
            dump_desc(p, b).wait()

        def dump_scatter(p, b):
            roff = s * ZSS + p * DS
            pltpu.sync_copy(acc.at[pl.ds(roff, DS), :], rows[b])
            pltpu.sync_copy(rows[b], out_hbm.at[dv[b]])

        dump_start(0, 0)

        def dpair(g, _):
            for b, joff in ((0, 0), (1, 1)):
                p = 2 * g + joff
                dump_wait(p, b)
                dump_start(p + 1, 1 - b)
                dump_scatter(p, b)
            return 0

        lax.fori_loop(0, DMP // 2 - 1, dpair, 0)
        dump_wait(DMP - 2, 0)
        dump_start(DMP - 1, 1)
        dump_scatter(DMP - 2, 0)
        dump_wait(DMP - 1, 1)
        dump_scatter(DMP - 1, 1)
        plsc.subcore_barrier()
        return 0

    lax.fori_loop(0, rq, round_body, 0)


def _sc_conv(tab, gidx, sidx, oidx, zeros_hbm, q):
    f = pl.kernel(
        functools.partial(_conv_body, q),
        out_type=jax.ShapeDtypeStruct((q * ACC_N, 8), jnp.float32),
        mesh=_mesh(),
        scratch_types=[
            pltpu.VMEM((K_EDGE,), jnp.int32),
            pltpu.VMEM((K_EDGE,), jnp.int32),
            pltpu.VMEM((K_EDGE,), jnp.int32),
            pltpu.VMEM((K_EDGE,), jnp.int32),
            pltpu.VMEM((K_EDGE, 8), jnp.float32),
            pltpu.VMEM((K_EDGE, 8), jnp.float32),
            pltpu.VMEM((DS,), jnp.int32),
            pltpu.VMEM((DS,), jnp.int32),
            pltpu.SemaphoreType.DMA,
            pltpu.SemaphoreType.DMA,
            pltpu.SemaphoreType.DMA,
            pltpu.SemaphoreType.DMA,
            pltpu.VMEM_SHARED((ACC_N, 8), jnp.float32),
        ],
        compiler_params=pltpu.CompilerParams(use_tc_tiling_on_sc=False),
    )
    return f(tab, gidx, sidx, oidx, zeros_hbm)


def _mm_body(x_ref, w_ref, o_ref):
    o_ref[...] = jnp.dot(x_ref[...], w_ref[...], preferred_element_type=jnp.float32)


def _mm(a, w):
    n, k = a.shape
    m = w.shape[1]
    bn = 2048
    return pl.pallas_call(
        _mm_body,
        grid=(pl.cdiv(n, bn),),
        in_specs=[
            pl.BlockSpec((bn, k), lambda i: (i, 0)),
            pl.BlockSpec((k, m), lambda i: (0, 0)),
        ],
        out_specs=pl.BlockSpec((bn, m), lambda i: (i, 0)),
        out_shape=jax.ShapeDtypeStruct((n, m), jnp.float32),
    )(a, w)


def _gcn_agg(y, gidx, sidx, oidx, zeros_hbm, q):
    tab = y.reshape(q * N_NODES, 8)
    outf = _sc_conv(tab, gidx, sidx, oidx, zeros_hbm, q)
    return outf.reshape(q * ACC_N * 8)[:N_NODES * q * 8].reshape(N_NODES, q * 8)


def kernel(x, edge_index, avg, mask, W_pre, b_pre, Ws_res, bs_res, W_final, b_final):
    src = edge_index[0]
    dst = edge_index[1]
    spad = jnp.pad(src, (0, E_PAD - E))
    dpad = jnp.pad(dst, (0, E_PAD - E), constant_values=GARB)
    dmask = jnp.where(spad == dpad, GARB, dpad)
    qs4 = jnp.arange(4, dtype=jnp.int32)
    qs6 = jnp.arange(6, dtype=jnp.int32)
    gidx4 = ((spad * 4)[None, :] + qs4[:, None]).reshape(-1)
    gidx6 = ((spad * 6)[None, :] + qs6[:, None]).reshape(-1)
    ar = jnp.arange(ACC_N, dtype=jnp.int32)
    oidx4 = ((ar * 4)[None, :] + qs4[:, None]).reshape(-1)
    oidx6 = ((ar * 6)[None, :] + qs6[:, None]).reshape(-1)
    zeros_hbm = jnp.zeros((DS, 8), jnp.float32)

    parts = _sc_degrees(dpad, dmask)
    deg_all = parts[0, :N_NODES] + parts[2, :N_NODES] + 1.0
    deg_keep = parts[1, :N_NODES] + parts[3, :N_NODES] + 1.0
    dinv_a = (deg_all ** -0.5)[:, None]
    dinv_k = (deg_keep ** -0.5)[:, None]

    xw = _mm(x, W_pre)
    y = dinv_k * xw
    agg = _gcn_agg(y, gidx4, dmask, oidx4, zeros_hbm, 4)
    x1 = dinv_k * agg + dinv_k * y + b_pre

    h = jnp.where(x != 0, x, x1)
    for i in range(DEPTH):
        xw = _mm(h, Ws_res[i])
        y = dinv_a * xw
        agg = _gcn_agg(y, gidx4, dpad, oidx4, zeros_hbm, 4)
        h = jax.nn.relu(dinv_a * agg + dinv_a * y + bs_res[i]) + h

    hc = jnp.concatenate([h, x], axis=1)
    xw = _mm(hc, W_final)
    y = dinv_a * xw
    agg = _gcn_agg(y, gidx6, dpad, oidx6, zeros_hbm, 6)
    out = dinv_a * agg + dinv_a * y + b_final
    out = jax.nn.sigmoid(out) * 255.0
    return (out * mask[:, None])[None]


# speedup vs baseline: 13.8849x; 1.0325x over previous
import functools

import jax
import jax.numpy as jnp
from jax import lax
from jax.experimental import pallas as pl
from jax.experimental.pallas import tpu as pltpu
from jax.experimental.pallas import tpu_sc as plsc

N_NODES = 215820
D_IN = 32
DEPTH = 5
OUT_CH = 48

NC = 2
NS = 16
NW = NC * NS
L = 16

E = 1726560
K_EDGE = 848
K_DEG = 4 * K_EDGE
CPW = 16
E_PAD = NW * K_DEG * CPW
CPS = E_PAD // (NS * K_EDGE)
ACC_N = 217088
GARB = ACC_N - 1
ZSS = ACC_N // NS
DS = K_EDGE
DMP = ZSS // DS


def _mesh():
    return plsc.VectorSubcoreMesh(core_axis_name="c", subcore_axis_name="s",
                                  num_cores=NC, num_subcores=NS)


def _deg_body(dst_hbm, dmask_hbm, out_hbm, d0, d1, m0, m1, ones_v, zero_v,
              isem0, isem1, acc_a, acc_k):
    c = lax.axis_index("c")
    s = lax.axis_index("s")
    wid = s * NC + c
    dbuf = (d0, d1)
    mbuf = (m0, m1)
    isem = (isem0, isem1)

    def ofill(i, _):
        ones_v[pl.ds(i * L, L)] = jnp.ones((L,), jnp.float32)
        return 0

    lax.fori_loop(0, K_DEG // L, ofill, 0)

    def zfill(i, _):
        zero_v[pl.ds(i * L, L)] = jnp.zeros((L,), jnp.float32)
        return 0

    lax.fori_loop(0, ZSS // L, zfill, 0)
    pltpu.sync_copy(zero_v, acc_a.at[pl.ds(s * ZSS, ZSS)])
    pltpu.sync_copy(zero_v, acc_k.at[pl.ds(s * ZSS, ZSS)])
    plsc.subcore_barrier()

    def idx_descs(j, b):
        base = (wid * CPW + j) * K_DEG
        return (
            pltpu.make_async_copy(dst_hbm.at[pl.ds(base, K_DEG)], dbuf[b], isem[b]),
            pltpu.make_async_copy(dmask_hbm.at[pl.ds(base, K_DEG)], mbuf[b], isem[b]),
        )

    def start_idx(j, b):
        for d in idx_descs(j, b):
            d.start()

    def wait_idx(j, b):
        for d in idx_descs(j, b):
            d.wait()

    def scatter(b):
        pltpu.sync_copy(ones_v, acc_a.at[dbuf[b]], add=True)
        pltpu.sync_copy(ones_v, acc_k.at[mbuf[b]], add=True)

    start_idx(0, 0)

    def pair(g, _):
        for b, joff in ((0, 0), (1, 1)):
            j = 2 * g + joff
            wait_idx(j, b)
            start_idx(j + 1, 1 - b)
            scatter(b)
        return 0

    lax.fori_loop(0, CPW // 2 - 1, pair, 0)
    wait_idx(CPW - 2, 0)
    start_idx(CPW - 1, 1)
    scatter(0)
    wait_idx(CPW - 1, 1)
    scatter(1)

    plsc.subcore_barrier()
    sl = pl.ds(s * ZSS, ZSS)
    pltpu.sync_copy(acc_a.at[sl], zero_v)
    pltpu.sync_copy(zero_v, out_hbm.at[pl.ds((c * 2) * ACC_N + s * ZSS, ZSS)])
    pltpu.sync_copy(acc_k.at[sl], zero_v)
    pltpu.sync_copy(zero_v, out_hbm.at[pl.ds((c * 2 + 1) * ACC_N + s * ZSS, ZSS)])


def _sc_degrees(dpad, dmask):
    f = pl.kernel(
        _deg_body,
        out_type=jax.ShapeDtypeStruct((NC * 2 * ACC_N,), jnp.float32),
        mesh=_mesh(),
        scratch_types=[
            pltpu.VMEM((K_DEG,), jnp.int32),
            pltpu.VMEM((K_DEG,), jnp.int32),
            pltpu.VMEM((K_DEG,), jnp.int32),
            pltpu.VMEM((K_DEG,), jnp.int32),
            pltpu.VMEM((K_DEG,), jnp.float32),
            pltpu.VMEM((ZSS,), jnp.float32),
            pltpu.SemaphoreType.DMA,
            pltpu.SemaphoreType.DMA,
            pltpu.VMEM_SHARED((ACC_N,), jnp.float32),
            pltpu.VMEM_SHARED((ACC_N,), jnp.float32),
        ],
        compiler_params=pltpu.CompilerParams(use_tc_tiling_on_sc=False),
    )
    return f(dpad, dmask).reshape(NC * 2, ACC_N)


def _conv_body(q, tab_hbm, gidx_hbm, sidx_hbm, oidx_hbm, zeros_hbm, out_hbm,
               gv0, gv1, sv0, sv1, rows0, rows1, dv0, dv1,
               isem0, isem1, gsem0, gsem1, acc):
    c = lax.axis_index("c")
    s = lax.axis_index("s")
    rq = q // NC
    gv = (gv0, gv1)
    sv = (sv0, sv1)
    rows = (rows0, rows1)
    isem = (isem0, isem1)
    gsem = (gsem0, gsem1)

    def round_body(r, _):
        qi = r * NC + c
        pltpu.sync_copy(zeros_hbm, rows0)

        def zcopy(p, _):
            pltpu.sync_copy(rows0, acc.at[pl.ds(s * ZSS + p * DS, DS), :])
            return 0

        lax.fori_loop(0, DMP, zcopy, 0)
        plsc.subcore_barrier()

        def idx_descs(j, b):
            base = (s * CPS + j) * K_EDGE
            return (
                pltpu.make_async_copy(
                    gidx_hbm.at[pl.ds(qi * E_PAD + base, K_EDGE)], gv[b], isem[b]),
                pltpu.make_async_copy(
                    sidx_hbm.at[pl.ds(base, K_EDGE)], sv[b], isem[b]),
            )

        def start_idx(j, b):
            for d in idx_descs(j, b):
                d.start()

        def wait_idx(j, b):
            for d in idx_descs(j, b):
                d.wait()

        def gather_desc(b):
            return pltpu.make_async_copy(tab_hbm.at[gv[b]], rows[b], gsem[b])

        def scatter(b):
            pltpu.sync_copy(rows[b], acc.at[sv[b]], add=True)

        start_idx(0, 0)
        start_idx(1, 1)
        wait_idx(0, 0)
        gather_desc(0).start()

        def pair(g, _):
            for b, joff in ((1, 1), (0, 2)):
                j = 2 * g + joff
                wait_idx(j, b)
                gather_desc(b).start()
                gather_desc(1 - b).wait()
                scatter(1 - b)
                start_idx(j + 1, 1 - b)

            return 0

        lax.fori_loop(0, (CPS - 2) // 2, pair, 0)

        wait_idx(CPS - 1, 1)
        gather_desc(1).start()
        gather_desc(0).wait()
        scatter(0)
        gather_desc(1).wait()
        scatter(1)
        plsc.subcore_barrier()

        dv = (dv0, dv1)

        def dump_desc(p, b):
            roff = s * ZSS + p * DS
            return pltpu.make_async_copy(
                oidx_hbm.at[pl.ds(qi * ACC_N + roff, DS)], dv[b], isem[b])

        def dump_start(p, b):
            dump_desc(p, b).start()

        def dump_wait(p, b):
            dump_desc(p, b).wait()

        def dump_scatter(p, b):
            roff = s * ZSS + p * DS
            pltpu.sync_copy(acc.at[pl.ds(roff, DS), :], rows[b])
            pltpu.sync_copy(rows[b], out_hbm.at[dv[b]])

        dump_start(0, 0)

        def dpair(g, _):
            for b, joff in ((0, 0), (1, 1)):
                p = 2 * g + joff
                dump_wait(p, b)
                dump_start(p + 1, 1 - b)
                dump_scatter(p, b)
            return 0

        lax.fori_loop(0, DMP // 2 - 1, dpair, 0)
        dump_wait(DMP - 2, 0)
        dump_start(DMP - 1, 1)
        dump_scatter(DMP - 2, 0)
        dump_wait(DMP - 1, 1)
        dump_scatter(DMP - 1, 1)
        plsc.subcore_barrier()
        return 0

    lax.fori_loop(0, rq, round_body, 0)


def _sc_conv(tab, gidx, sidx, oidx, zeros_hbm, q):
    f = pl.kernel(
        functools.partial(_conv_body, q),
        out_type=jax.ShapeDtypeStruct((q * ACC_N, 8), jnp.float32),
        mesh=_mesh(),
        scratch_types=[
            pltpu.VMEM((K_EDGE,), jnp.int32),
            pltpu.VMEM((K_EDGE,), jnp.int32),
            pltpu.VMEM((K_EDGE,), jnp.int32),
            pltpu.VMEM((K_EDGE,), jnp.int32),
            pltpu.VMEM((K_EDGE, 8), jnp.float32),
            pltpu.VMEM((K_EDGE, 8), jnp.float32),
            pltpu.VMEM((DS,), jnp.int32),
            pltpu.VMEM((DS,), jnp.int32),
            pltpu.SemaphoreType.DMA,
            pltpu.SemaphoreType.DMA,
            pltpu.SemaphoreType.DMA,
            pltpu.SemaphoreType.DMA,
            pltpu.VMEM_SHARED((ACC_N, 8), jnp.float32),
        ],
        compiler_params=pltpu.CompilerParams(use_tc_tiling_on_sc=False),
    )
    return f(tab, gidx, sidx, oidx, zeros_hbm)


def _mm_body(x_ref, w_ref, d_ref, o_ref):
    xw = jnp.dot(x_ref[...], w_ref[...], preferred_element_type=jnp.float32)
    o_ref[...] = xw * d_ref[...]


def _mm_scaled(a, w, dinv):
    n, k = a.shape
    m = w.shape[1]
    bn = 2048
    return pl.pallas_call(
        _mm_body,
        grid=(pl.cdiv(n, bn),),
        in_specs=[
            pl.BlockSpec((bn, k), lambda i: (i, 0)),
            pl.BlockSpec((k, m), lambda i: (0, 0)),
            pl.BlockSpec((bn, 1), lambda i: (i, 0)),
        ],
        out_specs=pl.BlockSpec((bn, m), lambda i: (i, 0)),
        out_shape=jax.ShapeDtypeStruct((n, m), jnp.float32),
    )(a, w, dinv)


def _gcn_agg(y, gidx, sidx, oidx, zeros_hbm, q):
    tab = y.reshape(q * N_NODES, 8)
    outf = _sc_conv(tab, gidx, sidx, oidx, zeros_hbm, q)
    return outf.reshape(q * ACC_N * 8)[:N_NODES * q * 8].reshape(N_NODES, q * 8)


def kernel(x, edge_index, avg, mask, W_pre, b_pre, Ws_res, bs_res, W_final, b_final):
    src = edge_index[0]
    dst = edge_index[1]
    spad = jnp.pad(src, (0, E_PAD - E))
    dpad = jnp.pad(dst, (0, E_PAD - E), constant_values=GARB)
    dmask = jnp.where(spad == dpad, GARB, dpad)
    qs4 = jnp.arange(4, dtype=jnp.int32)
    qs6 = jnp.arange(6, dtype=jnp.int32)
    gidx4 = ((spad * 4)[None, :] + qs4[:, None]).reshape(-1)
    gidx6 = ((spad * 6)[None, :] + qs6[:, None]).reshape(-1)
    ar = jnp.arange(ACC_N, dtype=jnp.int32)
    oidx4 = ((ar * 4)[None, :] + qs4[:, None]).reshape(-1)
    oidx6 = ((ar * 6)[None, :] + qs6[:, None]).reshape(-1)
    zeros_hbm = jnp.zeros((DS, 8), jnp.float32)

    parts = _sc_degrees(dpad, dmask)
    deg_all = parts[0, :N_NODES] + parts[2, :N_NODES] + 1.0
    deg_keep = parts[1, :N_NODES] + parts[3, :N_NODES] + 1.0
    dinv_a = (deg_all ** -0.5)[:, None]
    dinv_k = (deg_keep ** -0.5)[:, None]

    y = _mm_scaled(x, W_pre, dinv_k)
    agg = _gcn_agg(y, gidx4, dmask, oidx4, zeros_hbm, 4)
    x1 = dinv_k * (agg + y) + b_pre

    h = jnp.where(x != 0, x, x1)
    for i in range(DEPTH):
        y = _mm_scaled(h, Ws_res[i], dinv_a)
        agg = _gcn_agg(y, gidx4, dpad, oidx4, zeros_hbm, 4)
        h = jax.nn.relu(dinv_a * (agg + y) + bs_res[i]) + h

    hc = jnp.concatenate([h, x], axis=1)
    y = _mm_scaled(hc, W_final, dinv_a)
    agg = _gcn_agg(y, gidx6, dpad, oidx6, zeros_hbm, 6)
    out = dinv_a * (agg + y) + b_final
    out = jax.nn.sigmoid(out) * 255.0
    return (out * mask[:, None])[None]
